# baseline (speedup 1.0000x reference)
import functools

import jax
import jax.numpy as jnp
import numpy as np
from jax import lax
from jax.experimental import pallas as pl
from jax.experimental.pallas import tpu as pltpu
from jax.experimental.pallas import tpu_sc as plsc

_N = 10000
_E = 320000
_HID = 128
_H = 8
_C = 16
_ED = 16
_ROW = 144
_NTILES = 32
_EPW = _E // _NTILES
_K = 40
_NCHUNK = _EPW // _K
_NP = 10240
_RPT = _NP // 16
_RB = 32


def _node_body(x_ref, w_ref, asf_ref, adf_ref, sel_ref, h_ref, sa_ref, sd_ref):
    x = x_ref[...]
    h = jnp.dot(x, w_ref[...], preferred_element_type=jnp.float32)
    h_ref[...] = h
    sel = sel_ref[...]
    sa_ref[...] = jnp.dot(h * asf_ref[...], sel, preferred_element_type=jnp.float32)
    sd_ref[...] = jnp.dot(h * adf_ref[...], sel, preferred_element_type=jnp.float32)


def _edge_logit_body(ea_ref, we_ref, aef_ref, sel_ref, ae_ref):
    amat = jnp.dot(we_ref[...] * aef_ref[...], sel_ref[...],
                   preferred_element_type=jnp.float32)
    ae_ref[...] = jnp.dot(ea_ref[...], amat, preferred_element_type=jnp.float32)


_mesh = plsc.VectorSubcoreMesh(core_axis_name="c", subcore_axis_name="s")


_MEGA = 50
_NMEGA = _EPW // (_MEGA * _K)
_IDXROWS = 2 * _E // _K


@functools.partial(
    pl.kernel,
    out_type=jax.ShapeDtypeStruct((2, _NP, _ROW), jnp.float32),
    mesh=_mesh,
    compiler_params=pltpu.CompilerParams(use_tc_tiling_on_sc=False),
    scratch_types=[
        pltpu.VMEM((_MEGA, _K), jnp.int32),
        pltpu.VMEM((_MEGA, _K), jnp.int32),
        pltpu.VMEM((_K, 16), jnp.float32),
        pltpu.VMEM((_K, 16), jnp.float32),
        pltpu.VMEM((_K, 16), jnp.float32),
        pltpu.VMEM((_K, 16), jnp.float32),
        pltpu.VMEM((_K, 16), jnp.float32),
        pltpu.VMEM((_K, 16), jnp.float32),
        pltpu.VMEM((_K, _HID), jnp.float32),
        pltpu.VMEM((_K, _HID), jnp.float32),
        pltpu.VMEM((_K, _ROW), jnp.float32),
        pltpu.VMEM((_K, _ROW), jnp.float32),
        pltpu.VMEM((_RB, _ROW), jnp.float32),
        pltpu.VMEM_SHARED((_NP, _ROW), jnp.float32),
        pltpu.SemaphoreType.DMA,
        pltpu.SemaphoreType.DMA,
        pltpu.SemaphoreType.DMA,
        pltpu.SemaphoreType.DMA,
    ],
)
def _edge_pass(eidx, sa, sd, ae, h, out,
               idxs, idxd, ga0, ga1, gb0, gb1, aev0, aev1,
               hg0, hg1, mb0, mb1, zb, acc, semg0, semg1, sems0, sems1):
    c = lax.axis_index("c")
    s = lax.axis_index("s")
    gwid = c * 16 + s
    ga = (ga0, ga1)
    gb = (gb0, gb1)
    aev = (aev0, aev1)
    hg = (hg0, hg1)
    mb = (mb0, mb1)
    semg = (semg0, semg1)
    sems = (sems0, sems1)
    zeros16 = jnp.zeros((16,), jnp.float32)

    def _zrow(r, carry):
        for cc in range(_ROW // 16):
            zb[r, pl.ds(cc * 16, 16)] = zeros16
        return carry

    lax.fori_loop(0, _RB, _zrow, 0)
    r0 = s * _RPT
    for k in range(_RPT // _RB):
        pltpu.sync_copy(zb, acc.at[pl.ds(r0 + k * _RB, _RB)])
    plsc.subcore_barrier()

    row_base = gwid * (_EPW // _K)

    def _stage(m, jj, p):
        base = (row_base + m * _MEGA + jj) * _K
        pltpu.async_copy(sa.at[idxs.at[jj]], ga[p], semg[p])
        pltpu.async_copy(sd.at[idxd.at[jj]], gb[p], semg[p])
        pltpu.async_copy(h.at[idxs.at[jj]], hg[p], semg[p])
        pltpu.async_copy(ae.at[pl.ds(base, _K)], aev[p], semg[p])

    def _wait_gathers(p):
        pltpu.make_async_copy(sa.at[idxs.at[0]], ga[p], semg[p]).wait()
        pltpu.make_async_copy(sd.at[idxd.at[0]], gb[p], semg[p]).wait()
        pltpu.make_async_copy(h.at[idxs.at[0]], hg[p], semg[p]).wait()
        pltpu.make_async_copy(ae.at[pl.ds(0, _K)], aev[p], semg[p]).wait()

    def _scatter(jj, p):
        pltpu.async_copy(mb[p], acc.at[idxd.at[jj]], sems[p], add=True)

    def _wait_scatter(p):
        pltpu.make_async_copy(mb[p], acc.at[idxd.at[0]], sems[p]).wait()

    def _compute(p):
        def _edge(j, carry2):
            v = ga[p][j] + gb[p][j] + aev[p][j]
            v = jnp.where(v >= 0.0, v, 0.2 * v)
            w = jnp.exp(v)
            mb[p][j, pl.ds(_HID, 16)] = w
            for hh in range(_H):
                lane = jnp.full((16,), hh, jnp.int32)
                wsplat = w.at[lane].get(mode="promise_in_bounds")
                mb[p][j, pl.ds(hh * 16, 16)] = (
                    hg[p][j, pl.ds(hh * 16, 16)] * wsplat)
            return carry2

        lax.fori_loop(0, _K, _edge, 0, unroll=8)

    for m in range(_NMEGA):
        r = row_base + m * _MEGA
        pltpu.sync_copy(eidx.at[pl.ds(r, _MEGA)], idxs)
        pltpu.sync_copy(eidx.at[pl.ds(_IDXROWS // 2 + r, _MEGA)], idxd)
        _stage(m, 0, 0)

        def _pair(u, carry):
            jj = 2 * u
            _wait_gathers(0)

            @pl.when(u > 0)
            def _():
                _wait_scatter(1)

            _stage(m, jj + 1, 1)
            _compute(0)
            _scatter(jj, 0)

            _wait_gathers(1)
            _wait_scatter(0)

            @pl.when(u < _MEGA // 2 - 1)
            def _():
                _stage(m, jj + 2, 0)

            _compute(1)
            _scatter(jj + 1, 1)
            return carry

        lax.fori_loop(0, _MEGA // 2, _pair, 0)
        _wait_scatter(1)

    plsc.subcore_barrier()
    for k in range(_RPT // _RB):
        pltpu.sync_copy(acc.at[pl.ds(r0 + k * _RB, _RB)], zb)
        pltpu.sync_copy(zb, out.at[c, pl.ds(r0 + k * _RB, _RB)])


def _final_body(x_ref, acc_ref, bias_ref, g_ref, b_ref, rep_ref, o_ref):
    a0 = acc_ref[0]
    a1 = acc_ref[1]
    msg = a0[:, :_HID] + a1[:, :_HID]
    den = a0[:, _HID:_HID + _H] + a1[:, _HID:_HID + _H]
    den128 = jnp.dot(den, rep_ref[...], preferred_element_type=jnp.float32)
    y = x_ref[...] + msg / (den128 + 1e-16) + bias_ref[...]
    mu = jnp.mean(y, axis=-1, keepdims=True)
    d = y - mu
    var = jnp.mean(d * d, axis=-1, keepdims=True)
    o_ref[...] = d / jnp.sqrt(var + 1e-5) * g_ref[...] + b_ref[...]


def kernel(x, edge_index, edge_attr, W, att_src, att_dst, W_e, att_edge, bias,
           ln_g, ln_b):
    f32 = jnp.float32
    sel = jnp.kron(jnp.eye(_H, dtype=f32), jnp.ones((_C, 1), f32))
    sel = jnp.pad(sel, ((0, 0), (0, 16 - _H)))
    rep = jnp.kron(jnp.eye(_H, dtype=f32), jnp.ones((1, _C), f32))
    asf = att_src.reshape(1, _HID)
    adf = att_dst.reshape(1, _HID)
    aef = att_edge.reshape(1, _HID)

    bn = 1000
    h, sa, sd = pl.pallas_call(
        _node_body,
        grid=(_N // bn,),
        in_specs=[
            pl.BlockSpec((bn, _HID), lambda i: (i, 0)),
            pl.BlockSpec((_HID, _HID), lambda i: (0, 0)),
            pl.BlockSpec((1, _HID), lambda i: (0, 0)),
            pl.BlockSpec((1, _HID), lambda i: (0, 0)),
            pl.BlockSpec((_HID, 16), lambda i: (0, 0)),
        ],
        out_specs=[
            pl.BlockSpec((bn, _HID), lambda i: (i, 0)),
            pl.BlockSpec((bn, 16), lambda i: (i, 0)),
            pl.BlockSpec((bn, 16), lambda i: (i, 0)),
        ],
        out_shape=[
            jax.ShapeDtypeStruct((_N, _HID), f32),
            jax.ShapeDtypeStruct((_N, 16), f32),
            jax.ShapeDtypeStruct((_N, 16), f32),
        ],
    )(x, W, asf, adf, sel)

    be = 8000
    ae = pl.pallas_call(
        _edge_logit_body,
        grid=(_E // be,),
        in_specs=[
            pl.BlockSpec((be, _ED), lambda i: (i, 0)),
            pl.BlockSpec((_ED, _HID), lambda i: (0, 0)),
            pl.BlockSpec((1, _HID), lambda i: (0, 0)),
            pl.BlockSpec((_HID, 16), lambda i: (0, 0)),
        ],
        out_specs=pl.BlockSpec((be, 16), lambda i: (i, 0)),
        out_shape=jax.ShapeDtypeStruct((_E, 16), f32),
    )(edge_attr, W_e, aef, sel)

    acc = _edge_pass(edge_index.reshape(_IDXROWS, _K), sa, sd, ae, h)

    bf = 1000
    y = pl.pallas_call(
        _final_body,
        grid=(_N // bf,),
        in_specs=[
            pl.BlockSpec((bf, _HID), lambda i: (i, 0)),
            pl.BlockSpec((2, bf, _ROW), lambda i: (0, i, 0)),
            pl.BlockSpec((1, _HID), lambda i: (0, 0)),
            pl.BlockSpec((1, _HID), lambda i: (0, 0)),
            pl.BlockSpec((1, _HID), lambda i: (0, 0)),
            pl.BlockSpec((_H, _HID), lambda i: (0, 0)),
        ],
        out_specs=pl.BlockSpec((bf, _HID), lambda i: (i, 0)),
        out_shape=jax.ShapeDtypeStruct((_N, _HID), f32),
    )(x, acc, bias.reshape(1, _HID), ln_g.reshape(1, _HID),
      ln_b.reshape(1, _HID), rep)
    return y


# speedup vs baseline: 1.0329x; 1.0329x over previous
import functools

import jax
import jax.numpy as jnp
import numpy as np
from jax import lax
from jax.experimental import pallas as pl
from jax.experimental.pallas import tpu as pltpu
from jax.experimental.pallas import tpu_sc as plsc

_N = 10000
_E = 320000
_HID = 128
_H = 8
_C = 16
_ED = 16
_ROW = 144
_NTILES = 32
_EPW = _E // _NTILES
_K = 40
_NCHUNK = _EPW // _K
_NP = 10240
_RPT = _NP // 16
_RB = 32


def _node_body(x_ref, w_ref, asf_ref, adf_ref, sel_ref, h_ref, sa_ref, sd_ref):
    x = x_ref[...]
    h = jnp.dot(x, w_ref[...], preferred_element_type=jnp.float32)
    h_ref[...] = h
    sel = sel_ref[...]
    sa_ref[...] = jnp.dot(h * asf_ref[...], sel, preferred_element_type=jnp.float32)
    sd_ref[...] = jnp.dot(h * adf_ref[...], sel, preferred_element_type=jnp.float32)


def _edge_logit_body(ea_ref, we_ref, aef_ref, sel_ref, ae_ref):
    amat = jnp.dot(we_ref[...] * aef_ref[...], sel_ref[...],
                   preferred_element_type=jnp.float32)
    ae_ref[...] = jnp.dot(ea_ref[...], amat, preferred_element_type=jnp.float32)


_mesh = plsc.VectorSubcoreMesh(core_axis_name="c", subcore_axis_name="s")


_MEGA = 50
_NMEGA = _EPW // (_MEGA * _K)
_IDXROWS = 2 * _E // _K


@functools.partial(
    pl.kernel,
    out_type=jax.ShapeDtypeStruct((2, _NP, _ROW), jnp.float32),
    mesh=_mesh,
    compiler_params=pltpu.CompilerParams(use_tc_tiling_on_sc=False),
    scratch_types=[
        pltpu.VMEM((_MEGA, _K), jnp.int32),
        pltpu.VMEM((_MEGA, _K), jnp.int32),
        pltpu.VMEM((_K, 16), jnp.float32),
        pltpu.VMEM((_K, 16), jnp.float32),
        pltpu.VMEM((_K, 16), jnp.float32),
        pltpu.VMEM((_K, 16), jnp.float32),
        pltpu.VMEM((_K, 16), jnp.float32),
        pltpu.VMEM((_K, 16), jnp.float32),
        pltpu.VMEM((_K, _HID), jnp.float32),
        pltpu.VMEM((_K, _HID), jnp.float32),
        pltpu.VMEM((_K, _ROW), jnp.float32),
        pltpu.VMEM((_K, _ROW), jnp.float32),
        pltpu.VMEM((_RB, _ROW), jnp.float32),
        pltpu.VMEM_SHARED((_NP, _ROW), jnp.float32),
        pltpu.SemaphoreType.DMA,
        pltpu.SemaphoreType.DMA,
        pltpu.SemaphoreType.DMA,
        pltpu.SemaphoreType.DMA,
    ],
)
def _edge_pass(eidx, sa, sd, ae, h, out,
               idxs, idxd, ga0, ga1, gb0, gb1, aev0, aev1,
               hg0, hg1, mb0, mb1, zb, acc, semg0, semg1, sems0, sems1):
    c = lax.axis_index("c")
    s = lax.axis_index("s")
    gwid = c * 16 + s
    ga = (ga0, ga1)
    gb = (gb0, gb1)
    aev = (aev0, aev1)
    hg = (hg0, hg1)
    mb = (mb0, mb1)
    semg = (semg0, semg1)
    sems = (sems0, sems1)
    zeros16 = jnp.zeros((16,), jnp.float32)

    def _zrow(r, carry):
        for cc in range(_ROW // 16):
            zb[r, pl.ds(cc * 16, 16)] = zeros16
        return carry

    lax.fori_loop(0, _RB, _zrow, 0)
    r0 = s * _RPT
    for k in range(_RPT // _RB):
        pltpu.sync_copy(zb, acc.at[pl.ds(r0 + k * _RB, _RB)])
    plsc.subcore_barrier()

    row_base = gwid * (_EPW // _K)

    def _stage(m, jj, p):
        base = (row_base + m * _MEGA + jj) * _K
        pltpu.async_copy(sa.at[idxs.at[jj]], ga[p], semg[p])
        pltpu.async_copy(sd.at[idxd.at[jj]], gb[p], semg[p])
        pltpu.async_copy(h.at[idxs.at[jj]], hg[p], semg[p])
        pltpu.async_copy(ae.at[pl.ds(base, _K)], aev[p], semg[p])

    def _wait_gathers(p):
        pltpu.make_async_copy(sa.at[idxs.at[0]], ga[p], semg[p]).wait()
        pltpu.make_async_copy(sd.at[idxd.at[0]], gb[p], semg[p]).wait()
        pltpu.make_async_copy(h.at[idxs.at[0]], hg[p], semg[p]).wait()
        pltpu.make_async_copy(ae.at[pl.ds(0, _K)], aev[p], semg[p]).wait()

    def _scatter(jj, p):
        pltpu.async_copy(mb[p], acc.at[idxd.at[jj]], sems[p], add=True)

    def _wait_scatter(p):
        pltpu.make_async_copy(mb[p], acc.at[idxd.at[0]], sems[p]).wait()

    def _compute(p):
        def _edge(j, carry2):
            v = ga[p][j] + gb[p][j] + aev[p][j]
            v = jnp.where(v >= 0.0, v, 0.2 * v)
            w = jnp.exp(v)
            mb[p][j, pl.ds(_HID, 16)] = w
            for hh in range(_H):
                lane = jnp.full((16,), hh, jnp.int32)
                wsplat = w.at[lane].get(mode="promise_in_bounds")
                mb[p][j, pl.ds(hh * 16, 16)] = (
                    hg[p][j, pl.ds(hh * 16, 16)] * wsplat)
            return carry2

        lax.fori_loop(0, _K, _edge, 0, unroll=8)

    for m in range(_NMEGA):
        r = row_base + m * _MEGA
        pltpu.sync_copy(eidx.at[pl.ds(r, _MEGA)], idxs)
        pltpu.sync_copy(eidx.at[pl.ds(_IDXROWS // 2 + r, _MEGA)], idxd)
        _stage(m, 0, 0)

        def _pair(u, carry):
            jj = 2 * u
            _wait_gathers(0)

            @pl.when(u > 0)
            def _():
                _wait_scatter(1)

            _stage(m, jj + 1, 1)

            @pl.when(u > 0)
            def _():
                _wait_scatter(0)

            _compute(0)
            _scatter(jj, 0)
            _wait_gathers(1)

            @pl.when(u < _MEGA // 2 - 1)
            def _():
                _stage(m, jj + 2, 0)

            _compute(1)
            _scatter(jj + 1, 1)
            return carry

        lax.fori_loop(0, _MEGA // 2, _pair, 0)
        _wait_scatter(0)
        _wait_scatter(1)

    plsc.subcore_barrier()
    for k in range(_RPT // _RB):
        pltpu.sync_copy(acc.at[pl.ds(r0 + k * _RB, _RB)], zb)
        pltpu.sync_copy(zb, out.at[c, pl.ds(r0 + k * _RB, _RB)])


def _final_body(x_ref, acc_ref, bias_ref, g_ref, b_ref, rep_ref, o_ref):
    a0 = acc_ref[0]
    a1 = acc_ref[1]
    msg = a0[:, :_HID] + a1[:, :_HID]
    den = a0[:, _HID:_HID + _H] + a1[:, _HID:_HID + _H]
    den128 = jnp.dot(den, rep_ref[...], preferred_element_type=jnp.float32)
    y = x_ref[...] + msg / (den128 + 1e-16) + bias_ref[...]
    mu = jnp.mean(y, axis=-1, keepdims=True)
    d = y - mu
    var = jnp.mean(d * d, axis=-1, keepdims=True)
    o_ref[...] = d / jnp.sqrt(var + 1e-5) * g_ref[...] + b_ref[...]


def kernel(x, edge_index, edge_attr, W, att_src, att_dst, W_e, att_edge, bias,
           ln_g, ln_b):
    f32 = jnp.float32
    sel = jnp.kron(jnp.eye(_H, dtype=f32), jnp.ones((_C, 1), f32))
    sel = jnp.pad(sel, ((0, 0), (0, 16 - _H)))
    rep = jnp.kron(jnp.eye(_H, dtype=f32), jnp.ones((1, _C), f32))
    asf = att_src.reshape(1, _HID)
    adf = att_dst.reshape(1, _HID)
    aef = att_edge.reshape(1, _HID)

    bn = 1000
    h, sa, sd = pl.pallas_call(
        _node_body,
        grid=(_N // bn,),
        in_specs=[
            pl.BlockSpec((bn, _HID), lambda i: (i, 0)),
            pl.BlockSpec((_HID, _HID), lambda i: (0, 0)),
            pl.BlockSpec((1, _HID), lambda i: (0, 0)),
            pl.BlockSpec((1, _HID), lambda i: (0, 0)),
            pl.BlockSpec((_HID, 16), lambda i: (0, 0)),
        ],
        out_specs=[
            pl.BlockSpec((bn, _HID), lambda i: (i, 0)),
            pl.BlockSpec((bn, 16), lambda i: (i, 0)),
            pl.BlockSpec((bn, 16), lambda i: (i, 0)),
        ],
        out_shape=[
            jax.ShapeDtypeStruct((_N, _HID), f32),
            jax.ShapeDtypeStruct((_N, 16), f32),
            jax.ShapeDtypeStruct((_N, 16), f32),
        ],
    )(x, W, asf, adf, sel)

    be = 8000
    ae = pl.pallas_call(
        _edge_logit_body,
        grid=(_E // be,),
        in_specs=[
            pl.BlockSpec((be, _ED), lambda i: (i, 0)),
            pl.BlockSpec((_ED, _HID), lambda i: (0, 0)),
            pl.BlockSpec((1, _HID), lambda i: (0, 0)),
            pl.BlockSpec((_HID, 16), lambda i: (0, 0)),
        ],
        out_specs=pl.BlockSpec((be, 16), lambda i: (i, 0)),
        out_shape=jax.ShapeDtypeStruct((_E, 16), f32),
    )(edge_attr, W_e, aef, sel)

    acc = _edge_pass(edge_index.reshape(_IDXROWS, _K), sa, sd, ae, h)

    bf = 1000
    y = pl.pallas_call(
        _final_body,
        grid=(_N // bf,),
        in_specs=[
            pl.BlockSpec((bf, _HID), lambda i: (i, 0)),
            pl.BlockSpec((2, bf, _ROW), lambda i: (0, i, 0)),
            pl.BlockSpec((1, _HID), lambda i: (0, 0)),
            pl.BlockSpec((1, _HID), lambda i: (0, 0)),
            pl.BlockSpec((1, _HID), lambda i: (0, 0)),
            pl.BlockSpec((_H, _HID), lambda i: (0, 0)),
        ],
        out_specs=pl.BlockSpec((bf, _HID), lambda i: (i, 0)),
        out_shape=jax.ShapeDtypeStruct((_N, _HID), f32),
    )(x, acc, bias.reshape(1, _HID), ln_g.reshape(1, _HID),
      ln_b.reshape(1, _HID), rep)
    return y


# speedup vs baseline: 1.0367x; 1.0037x over previous
import functools

import jax
import jax.numpy as jnp
import numpy as np
from jax import lax
from jax.experimental import pallas as pl
from jax.experimental.pallas import tpu as pltpu
from jax.experimental.pallas import tpu_sc as plsc

_N = 10000
_E = 320000
_HID = 128
_H = 8
_C = 16
_ED = 16
_ROW = 144
_NTILES = 32
_EPW = _E // _NTILES
_K = 40
_NCHUNK = _EPW // _K
_NP = 10240
_RPT = _NP // 16
_RB = 32


def _dense_body(x_ref, ea_ref, w_ref, asf_ref, adf_ref, aef_ref, we_ref,
                sel_ref, hx_ref, sd_ref, ae_ref):
    x = x_ref[...]
    h = jnp.dot(x, w_ref[...], preferred_element_type=jnp.float32)
    sel = sel_ref[...]
    hx_ref[:, : _HID] = h
    hx_ref[:, _HID:] = jnp.dot(h * asf_ref[...], sel,
                               preferred_element_type=jnp.float32)
    sd_ref[...] = jnp.dot(h * adf_ref[...], sel,
                          preferred_element_type=jnp.float32)
    amat = jnp.dot(we_ref[...] * aef_ref[...], sel_ref[...],
                   preferred_element_type=jnp.float32)
    ae_ref[...] = jnp.dot(ea_ref[...], amat, preferred_element_type=jnp.float32)


_mesh = plsc.VectorSubcoreMesh(core_axis_name="c", subcore_axis_name="s")


_MEGA = 50
_NMEGA = _EPW // (_MEGA * _K)
_IDXROWS = 2 * _E // _K


@functools.partial(
    pl.kernel,
    out_type=jax.ShapeDtypeStruct((2, _NP, _ROW), jnp.float32),
    mesh=_mesh,
    compiler_params=pltpu.CompilerParams(use_tc_tiling_on_sc=False),
    scratch_types=[
        pltpu.VMEM((_MEGA, _K), jnp.int32),
        pltpu.VMEM((_MEGA, _K), jnp.int32),
        pltpu.VMEM((_K, 16), jnp.float32),
        pltpu.VMEM((_K, 16), jnp.float32),
        pltpu.VMEM((_K, 16), jnp.float32),
        pltpu.VMEM((_K, 16), jnp.float32),
        pltpu.VMEM((_K, _ROW), jnp.float32),
        pltpu.VMEM((_K, _ROW), jnp.float32),
        pltpu.VMEM((_K, _ROW), jnp.float32),
        pltpu.VMEM((_K, _ROW), jnp.float32),
        pltpu.VMEM((_RB, _ROW), jnp.float32),
        pltpu.VMEM_SHARED((_NP, _ROW), jnp.float32),
        pltpu.SemaphoreType.DMA,
        pltpu.SemaphoreType.DMA,
        pltpu.SemaphoreType.DMA,
        pltpu.SemaphoreType.DMA,
    ],
)
def _edge_pass(eidx, sd, ae, hx, out,
               idxs, idxd, gb0, gb1, aev0, aev1,
               hg0, hg1, mb0, mb1, zb, acc, semg0, semg1, sems0, sems1):
    c = lax.axis_index("c")
    s = lax.axis_index("s")
    gwid = c * 16 + s
    gb = (gb0, gb1)
    aev = (aev0, aev1)
    hg = (hg0, hg1)
    mb = (mb0, mb1)
    semg = (semg0, semg1)
    sems = (sems0, sems1)
    zeros16 = jnp.zeros((16,), jnp.float32)

    def _zrow(r, carry):
        for cc in range(_ROW // 16):
            zb[r, pl.ds(cc * 16, 16)] = zeros16
        return carry

    lax.fori_loop(0, _RB, _zrow, 0)
    r0 = s * _RPT
    for k in range(_RPT // _RB):
        pltpu.sync_copy(zb, acc.at[pl.ds(r0 + k * _RB, _RB)])
    plsc.subcore_barrier()

    row_base = gwid * (_EPW // _K)

    def _stage(m, jj, p):
        base = (row_base + m * _MEGA + jj) * _K
        pltpu.async_copy(sd.at[idxd.at[jj]], gb[p], semg[p])
        pltpu.async_copy(hx.at[idxs.at[jj]], hg[p], semg[p])
        pltpu.async_copy(ae.at[pl.ds(base, _K)], aev[p], semg[p])

    def _wait_gathers(p):
        pltpu.make_async_copy(sd.at[idxd.at[0]], gb[p], semg[p]).wait()
        pltpu.make_async_copy(hx.at[idxs.at[0]], hg[p], semg[p]).wait()
        pltpu.make_async_copy(ae.at[pl.ds(0, _K)], aev[p], semg[p]).wait()

    def _scatter(jj, p):
        pltpu.async_copy(mb[p], acc.at[idxd.at[jj]], sems[p], add=True)

    def _wait_scatter(p):
        pltpu.make_async_copy(mb[p], acc.at[idxd.at[0]], sems[p]).wait()

    def _compute(p):
        def _edge(j, carry2):
            v = hg[p][j, pl.ds(_HID, 16)] + gb[p][j] + aev[p][j]
            v = jnp.where(v >= 0.0, v, 0.2 * v)
            w = jnp.exp(v)
            mb[p][j, pl.ds(_HID, 16)] = w
            for hh in range(_H):
                lane = jnp.full((16,), hh, jnp.int32)
                wsplat = w.at[lane].get(mode="promise_in_bounds")
                mb[p][j, pl.ds(hh * 16, 16)] = (
                    hg[p][j, pl.ds(hh * 16, 16)] * wsplat)
            return carry2

        lax.fori_loop(0, _K, _edge, 0, unroll=8)

    for m in range(_NMEGA):
        r = row_base + m * _MEGA
        pltpu.sync_copy(eidx.at[pl.ds(r, _MEGA)], idxs)
        pltpu.sync_copy(eidx.at[pl.ds(_IDXROWS // 2 + r, _MEGA)], idxd)
        _stage(m, 0, 0)

        def _pair(u, carry):
            jj = 2 * u
            _wait_gathers(0)

            @pl.when(u > 0)
            def _():
                _wait_scatter(1)

            _stage(m, jj + 1, 1)

            @pl.when(u > 0)
            def _():
                _wait_scatter(0)

            _compute(0)
            _scatter(jj, 0)
            _wait_gathers(1)

            @pl.when(u < _MEGA // 2 - 1)
            def _():
                _stage(m, jj + 2, 0)

            _compute(1)
            _scatter(jj + 1, 1)
            return carry

        lax.fori_loop(0, _MEGA // 2, _pair, 0)
        _wait_scatter(0)
        _wait_scatter(1)

    plsc.subcore_barrier()
    for k in range(_RPT // _RB):
        pltpu.sync_copy(acc.at[pl.ds(r0 + k * _RB, _RB)], zb)
        pltpu.sync_copy(zb, out.at[c, pl.ds(r0 + k * _RB, _RB)])


def _final_body(x_ref, acc_ref, bias_ref, g_ref, b_ref, rep_ref, o_ref):
    a0 = acc_ref[0]
    a1 = acc_ref[1]
    msg = a0[:, :_HID] + a1[:, :_HID]
    den = a0[:, _HID:_HID + _H] + a1[:, _HID:_HID + _H]
    den128 = jnp.dot(den, rep_ref[...], preferred_element_type=jnp.float32)
    y = x_ref[...] + msg / (den128 + 1e-16) + bias_ref[...]
    mu = jnp.mean(y, axis=-1, keepdims=True)
    d = y - mu
    var = jnp.mean(d * d, axis=-1, keepdims=True)
    o_ref[...] = d / jnp.sqrt(var + 1e-5) * g_ref[...] + b_ref[...]


def kernel(x, edge_index, edge_attr, W, att_src, att_dst, W_e, att_edge, bias,
           ln_g, ln_b):
    f32 = jnp.float32
    sel = jnp.kron(jnp.eye(_H, dtype=f32), jnp.ones((_C, 1), f32))
    sel = jnp.pad(sel, ((0, 0), (0, 16 - _H)))
    rep = jnp.kron(jnp.eye(_H, dtype=f32), jnp.ones((1, _C), f32))
    asf = att_src.reshape(1, _HID)
    adf = att_dst.reshape(1, _HID)
    aef = att_edge.reshape(1, _HID)

    bn = 400
    bre = _E // (_N // bn)
    hx, sd, ae = pl.pallas_call(
        _dense_body,
        grid=(_N // bn,),
        in_specs=[
            pl.BlockSpec((bn, _HID), lambda i: (i, 0)),
            pl.BlockSpec((bre, _ED), lambda i: (i, 0)),
            pl.BlockSpec((_HID, _HID), lambda i: (0, 0)),
            pl.BlockSpec((1, _HID), lambda i: (0, 0)),
            pl.BlockSpec((1, _HID), lambda i: (0, 0)),
            pl.BlockSpec((1, _HID), lambda i: (0, 0)),
            pl.BlockSpec((_ED, _HID), lambda i: (0, 0)),
            pl.BlockSpec((_HID, 16), lambda i: (0, 0)),
        ],
        out_specs=[
            pl.BlockSpec((bn, _ROW), lambda i: (i, 0)),
            pl.BlockSpec((bn, 16), lambda i: (i, 0)),
            pl.BlockSpec((bre, 16), lambda i: (i, 0)),
        ],
        out_shape=[
            jax.ShapeDtypeStruct((_N, _ROW), f32),
            jax.ShapeDtypeStruct((_N, 16), f32),
            jax.ShapeDtypeStruct((_E, 16), f32),
        ],
    )(x, edge_attr, W, asf, adf, aef, W_e, sel)

    acc = _edge_pass(edge_index.reshape(_IDXROWS, _K), sd, ae, hx)

    bf = 1000
    y = pl.pallas_call(
        _final_body,
        grid=(_N // bf,),
        in_specs=[
            pl.BlockSpec((bf, _HID), lambda i: (i, 0)),
            pl.BlockSpec((2, bf, _ROW), lambda i: (0, i, 0)),
            pl.BlockSpec((1, _HID), lambda i: (0, 0)),
            pl.BlockSpec((1, _HID), lambda i: (0, 0)),
            pl.BlockSpec((1, _HID), lambda i: (0, 0)),
            pl.BlockSpec((_H, _HID), lambda i: (0, 0)),
        ],
        out_specs=pl.BlockSpec((bf, _HID), lambda i: (i, 0)),
        out_shape=jax.ShapeDtypeStruct((_N, _HID), f32),
    )(x, acc, bias.reshape(1, _HID), ln_g.reshape(1, _HID),
      ln_b.reshape(1, _HID), rep)
    return y


# speedup vs baseline: 1.2110x; 1.1681x over previous
import functools

import jax
import jax.numpy as jnp
import numpy as np
from jax import lax
from jax.experimental import pallas as pl
from jax.experimental.pallas import tpu as pltpu
from jax.experimental.pallas import tpu_sc as plsc

_N = 10000
_E = 320000
_HID = 128
_H = 8
_C = 16
_ED = 16
_ROW = 144
_NTILES = 32
_EPW = _E // _NTILES
_K = 40
_NCHUNK = _EPW // _K
_NP = 10240
_RPT = _NP // 16
_RB = 32


def _dense_body(x_ref, ea_ref, w_ref, asf_ref, adf_ref, aef_ref, we_ref,
                sel_ref, smat_ref, tmat_ref, mmat_ref, hx_ref, sd_ref, ae_ref):
    x = x_ref[...]
    h = jnp.dot(x, w_ref[...], preferred_element_type=jnp.float32)
    sel = sel_ref[...]
    hx_ref[:, : _HID] = h
    hx_ref[:, _HID:] = jnp.dot(h * asf_ref[...], sel,
                               preferred_element_type=jnp.float32)
    sd_ref[...] = jnp.dot(h * adf_ref[...], sel,
                          preferred_element_type=jnp.float32)
    amat = jnp.dot(we_ref[...] * aef_ref[...], sel_ref[...],
                   preferred_element_type=jnp.float32)
    bmat = jnp.dot(jnp.dot(smat_ref[...], amat, preferred_element_type=jnp.float32),
                   tmat_ref[...], preferred_element_type=jnp.float32) * mmat_ref[...]
    ae_ref[...] = jnp.dot(ea_ref[...], bmat, preferred_element_type=jnp.float32)


_mesh = plsc.VectorSubcoreMesh(core_axis_name="c", subcore_axis_name="s")


_MEGA = 50
_NMEGA = _EPW // (_MEGA * _K)
_IDXROWS = 2 * _E // _K


@functools.partial(
    pl.kernel,
    out_type=jax.ShapeDtypeStruct((2, _NP, _ROW), jnp.float32),
    mesh=_mesh,
    compiler_params=pltpu.CompilerParams(use_tc_tiling_on_sc=False),
    scratch_types=[
        pltpu.VMEM((_MEGA, _K), jnp.int32),
        pltpu.VMEM((_MEGA, _K), jnp.int32),
        pltpu.VMEM((_K, 16), jnp.float32),
        pltpu.VMEM((_K, 16), jnp.float32),
        pltpu.VMEM((_K * 16,), jnp.float32),
        pltpu.VMEM((_K * 16,), jnp.float32),
        pltpu.VMEM((_K, _ROW), jnp.float32),
        pltpu.VMEM((_K, _ROW), jnp.float32),
        pltpu.VMEM((_K, _ROW), jnp.float32),
        pltpu.VMEM((_K, _ROW), jnp.float32),
        pltpu.VMEM((_RB, _ROW), jnp.float32),
        pltpu.VMEM_SHARED((_NP, _ROW), jnp.float32),
        pltpu.SemaphoreType.DMA,
        pltpu.SemaphoreType.DMA,
        pltpu.SemaphoreType.DMA,
        pltpu.SemaphoreType.DMA,
    ],
)
def _edge_pass(eidx, sd, ae, hx, out,
               idxs, idxd, gb0, gb1, aev0, aev1,
               hg0, hg1, mb0, mb1, zb, acc, semg0, semg1, sems0, sems1):
    c = lax.axis_index("c")
    s = lax.axis_index("s")
    gwid = c * 16 + s
    gb = (gb0, gb1)
    aev = (aev0, aev1)
    hg = (hg0, hg1)
    mb = (mb0, mb1)
    semg = (semg0, semg1)
    sems = (sems0, sems1)
    zeros16 = jnp.zeros((16,), jnp.float32)

    def _zrow(r, carry):
        for cc in range(_ROW // 16):
            zb[r, pl.ds(cc * 16, 16)] = zeros16
        return carry

    lax.fori_loop(0, _RB, _zrow, 0)
    r0 = s * _RPT
    for k in range(_RPT // _RB):
        pltpu.sync_copy(zb, acc.at[pl.ds(r0 + k * _RB, _RB)])
    plsc.subcore_barrier()

    row_base = gwid * (_EPW // _K)

    def _stage(m, jj, p):
        base = (row_base + m * _MEGA + jj) * _K
        pltpu.async_copy(sd.at[idxd.at[jj]], gb[p], semg[p])
        pltpu.async_copy(hx.at[idxs.at[jj]], hg[p], semg[p])
        pltpu.async_copy(ae.at[pl.ds(base * 16, _K * 16)], aev[p], semg[p])

    def _wait_gathers(p):
        pltpu.make_async_copy(sd.at[idxd.at[0]], gb[p], semg[p]).wait()
        pltpu.make_async_copy(hx.at[idxs.at[0]], hg[p], semg[p]).wait()
        pltpu.make_async_copy(ae.at[pl.ds(0, _K * 16)], aev[p], semg[p]).wait()

    def _scatter(jj, p):
        pltpu.async_copy(mb[p], acc.at[idxd.at[jj]], sems[p], add=True)

    def _wait_scatter(p):
        pltpu.make_async_copy(mb[p], acc.at[idxd.at[0]], sems[p]).wait()

    def _compute(p):
        def _edge(j, carry2):
            v = hg[p][j, pl.ds(_HID, 16)] + gb[p][j] + aev[p][pl.ds(j * 16, 16)]
            v = jnp.where(v >= 0.0, v, 0.2 * v)
            w = jnp.exp(v)
            mb[p][j, pl.ds(_HID, 16)] = w
            for hh in range(_H):
                lane = jnp.full((16,), hh, jnp.int32)
                wsplat = w.at[lane].get(mode="promise_in_bounds")
                mb[p][j, pl.ds(hh * 16, 16)] = (
                    hg[p][j, pl.ds(hh * 16, 16)] * wsplat)
            return carry2

        lax.fori_loop(0, _K, _edge, 0, unroll=8)

    for m in range(_NMEGA):
        r = row_base + m * _MEGA
        pltpu.sync_copy(eidx.at[pl.ds(r, _MEGA)], idxs)
        pltpu.sync_copy(eidx.at[pl.ds(_IDXROWS // 2 + r, _MEGA)], idxd)
        _stage(m, 0, 0)

        def _pair(u, carry):
            jj = 2 * u
            _wait_gathers(0)

            @pl.when(u > 0)
            def _():
                _wait_scatter(1)

            _stage(m, jj + 1, 1)

            @pl.when(u > 0)
            def _():
                _wait_scatter(0)

            _compute(0)
            _scatter(jj, 0)
            _wait_gathers(1)

            @pl.when(u < _MEGA // 2 - 1)
            def _():
                _stage(m, jj + 2, 0)

            _compute(1)
            _scatter(jj + 1, 1)
            return carry

        lax.fori_loop(0, _MEGA // 2, _pair, 0)
        _wait_scatter(0)
        _wait_scatter(1)

    plsc.subcore_barrier()
    for k in range(_RPT // _RB):
        pltpu.sync_copy(acc.at[pl.ds(r0 + k * _RB, _RB)], zb)
        pltpu.sync_copy(zb, out.at[c, pl.ds(r0 + k * _RB, _RB)])


def _final_body(x_ref, acc_ref, bias_ref, g_ref, b_ref, rep_ref, o_ref):
    a0 = acc_ref[0]
    a1 = acc_ref[1]
    msg = a0[:, :_HID] + a1[:, :_HID]
    den = a0[:, _HID:_HID + _H] + a1[:, _HID:_HID + _H]
    den128 = jnp.dot(den, rep_ref[...], preferred_element_type=jnp.float32)
    y = x_ref[...] + msg / (den128 + 1e-16) + bias_ref[...]
    mu = jnp.mean(y, axis=-1, keepdims=True)
    d = y - mu
    var = jnp.mean(d * d, axis=-1, keepdims=True)
    o_ref[...] = d / jnp.sqrt(var + 1e-5) * g_ref[...] + b_ref[...]


def kernel(x, edge_index, edge_attr, W, att_src, att_dst, W_e, att_edge, bias,
           ln_g, ln_b):
    f32 = jnp.float32
    sel = jnp.kron(jnp.eye(_H, dtype=f32), jnp.ones((_C, 1), f32))
    sel = jnp.pad(sel, ((0, 0), (0, 16 - _H)))
    rep = jnp.kron(jnp.eye(_H, dtype=f32), jnp.ones((1, _C), f32))
    asf = att_src.reshape(1, _HID)
    adf = att_dst.reshape(1, _HID)
    aef = att_edge.reshape(1, _HID)

    smat = jnp.kron(jnp.ones((_H, 1), f32), jnp.eye(16, dtype=f32))
    tmat = jnp.kron(jnp.ones((1, _H), f32), jnp.eye(16, dtype=f32))
    mmat = jnp.kron(jnp.eye(_H, dtype=f32), jnp.ones((16, 16), f32))
    ea2 = edge_attr.reshape(_E // 8, 128)
    bn = 400
    brr = (_E // 8) // (_N // bn)
    hx, sd, ae2 = pl.pallas_call(
        _dense_body,
        grid=(_N // bn,),
        in_specs=[
            pl.BlockSpec((bn, _HID), lambda i: (i, 0)),
            pl.BlockSpec((brr, 128), lambda i: (i, 0)),
            pl.BlockSpec((_HID, _HID), lambda i: (0, 0)),
            pl.BlockSpec((1, _HID), lambda i: (0, 0)),
            pl.BlockSpec((1, _HID), lambda i: (0, 0)),
            pl.BlockSpec((1, _HID), lambda i: (0, 0)),
            pl.BlockSpec((_ED, _HID), lambda i: (0, 0)),
            pl.BlockSpec((_HID, 16), lambda i: (0, 0)),
            pl.BlockSpec((_HID, 16), lambda i: (0, 0)),
            pl.BlockSpec((16, _HID), lambda i: (0, 0)),
            pl.BlockSpec((_HID, _HID), lambda i: (0, 0)),
        ],
        out_specs=[
            pl.BlockSpec((bn, _ROW), lambda i: (i, 0)),
            pl.BlockSpec((bn, 16), lambda i: (i, 0)),
            pl.BlockSpec((brr, 128), lambda i: (i, 0)),
        ],
        out_shape=[
            jax.ShapeDtypeStruct((_N, _ROW), f32),
            jax.ShapeDtypeStruct((_N, 16), f32),
            jax.ShapeDtypeStruct((_E // 8, 128), f32),
        ],
    )(x, ea2, W, asf, adf, aef, W_e, sel, smat, tmat, mmat)
    ae = ae2.reshape(-1)

    acc = _edge_pass(edge_index.reshape(_IDXROWS, _K), sd, ae, hx)

    bf = 1000
    y = pl.pallas_call(
        _final_body,
        grid=(_N // bf,),
        in_specs=[
            pl.BlockSpec((bf, _HID), lambda i: (i, 0)),
            pl.BlockSpec((2, bf, _ROW), lambda i: (0, i, 0)),
            pl.BlockSpec((1, _HID), lambda i: (0, 0)),
            pl.BlockSpec((1, _HID), lambda i: (0, 0)),
            pl.BlockSpec((1, _HID), lambda i: (0, 0)),
            pl.BlockSpec((_H, _HID), lambda i: (0, 0)),
        ],
        out_specs=pl.BlockSpec((bf, _HID), lambda i: (i, 0)),
        out_shape=jax.ShapeDtypeStruct((_N, _HID), f32),
    )(x, acc, bias.reshape(1, _HID), ln_g.reshape(1, _HID),
      ln_b.reshape(1, _HID), rep)
    return y


# speedup vs baseline: 1.9457x; 1.6067x over previous
import functools

import jax
import jax.numpy as jnp
import numpy as np
from jax import lax
from jax.experimental import pallas as pl
from jax.experimental.pallas import tpu as pltpu
from jax.experimental.pallas import tpu_sc as plsc

_N = 10000
_E = 320000
_HID = 128
_H = 8
_C = 16
_ED = 16
_ROW = 144
_NTILES = 32
_EPW = _E // _NTILES
_K = 40
_NCHUNK = _EPW // _K
_NP = 10240
_RPT = _NP // 16
_RB = 32


def _dense_body(x_ref, ea_ref, w_ref, asf_ref, adf_ref, aef_ref, we_ref,
                sel_ref, smat_ref, tmat_ref, mmat_ref, hx_ref, sd_ref, ae_ref):
    x = x_ref[...]
    h = jnp.dot(x, w_ref[...], preferred_element_type=jnp.float32)
    sel = sel_ref[...]
    hx_ref[:, : _HID] = h
    hx_ref[:, _HID:] = jnp.dot(h * asf_ref[...], sel,
                               preferred_element_type=jnp.float32)
    sd_ref[...] = jnp.dot(h * adf_ref[...], sel,
                          preferred_element_type=jnp.float32)
    amat = jnp.dot(we_ref[...] * aef_ref[...], sel_ref[...],
                   preferred_element_type=jnp.float32)
    bmat = jnp.dot(jnp.dot(smat_ref[...], amat, preferred_element_type=jnp.float32),
                   tmat_ref[...], preferred_element_type=jnp.float32) * mmat_ref[...]
    ae_ref[...] = jnp.dot(ea_ref[...], bmat, preferred_element_type=jnp.float32)


_mesh = plsc.VectorSubcoreMesh(core_axis_name="c", subcore_axis_name="s")


_MEGA = 50
_NMEGA = _EPW // (_MEGA * _K)
_IDXROWS = 2 * _E // _K


@functools.partial(
    pl.kernel,
    out_type=jax.ShapeDtypeStruct((2, _NP, _ROW), jnp.float32),
    mesh=_mesh,
    compiler_params=pltpu.CompilerParams(use_tc_tiling_on_sc=False),
    scratch_types=[
        pltpu.VMEM((_MEGA, _K), jnp.int32),
        pltpu.VMEM((_MEGA, _K), jnp.int32),
        pltpu.VMEM((_K, 16), jnp.float32),
        pltpu.VMEM((_K, 16), jnp.float32),
        pltpu.VMEM((_K * 16,), jnp.float32),
        pltpu.VMEM((_K * 16,), jnp.float32),
        pltpu.VMEM((_K, _ROW), jnp.float32),
        pltpu.VMEM((_K, _ROW), jnp.float32),
        pltpu.VMEM((_K, _ROW), jnp.float32),
        pltpu.VMEM((_K, _ROW), jnp.float32),
        pltpu.VMEM((_RB, _ROW), jnp.float32),
        pltpu.VMEM_SHARED((_NP, _ROW), jnp.float32),
        pltpu.SemaphoreType.DMA,
        pltpu.SemaphoreType.DMA,
        pltpu.SemaphoreType.DMA,
        pltpu.SemaphoreType.DMA,
    ],
)
def _edge_pass(eidx, sd, ae, hx, out,
               idxs, idxd, gb0, gb1, aev0, aev1,
               hg0, hg1, mb0, mb1, zb, acc, semg0, semg1, sems0, sems1):
    c = lax.axis_index("c")
    s = lax.axis_index("s")
    gwid = c * 16 + s
    gb = (gb0, gb1)
    aev = (aev0, aev1)
    hg = (hg0, hg1)
    mb = (mb0, mb1)
    semg = (semg0, semg1)
    sems = (sems0, sems1)
    zeros16 = jnp.zeros((16,), jnp.float32)

    def _zrow(r, carry):
        for cc in range(_ROW // 16):
            zb[r, pl.ds(cc * 16, 16)] = zeros16
        return carry

    lax.fori_loop(0, _RB, _zrow, 0)
    r0 = s * _RPT
    for k in range(_RPT // _RB):
        pltpu.sync_copy(zb, acc.at[pl.ds(r0 + k * _RB, _RB)])
    plsc.subcore_barrier()

    row_base = gwid * (_EPW // _K)

    def _stage(m, jj, p):
        base = (row_base + m * _MEGA + jj) * _K
        pltpu.async_copy(sd.at[idxd.at[jj]], gb[p], semg[p])
        pltpu.async_copy(hx.at[idxs.at[jj]], hg[p], semg[p])
        pltpu.async_copy(ae.at[pl.ds(base * 16, _K * 16)], aev[p], semg[p])

    def _wait_gathers(p):
        pltpu.make_async_copy(sd.at[idxd.at[0]], gb[p], semg[p]).wait()
        pltpu.make_async_copy(hx.at[idxs.at[0]], hg[p], semg[p]).wait()
        pltpu.make_async_copy(ae.at[pl.ds(0, _K * 16)], aev[p], semg[p]).wait()

    def _scatter(jj, p):
        pltpu.async_copy(mb[p], acc.at[idxd.at[jj]], sems[p], add=True)

    def _wait_scatter(p):
        pltpu.make_async_copy(mb[p], acc.at[idxd.at[0]], sems[p]).wait()

    def _compute(p):
        @plsc.parallel_loop(0, _K, 1, unroll=8)
        def _edge(j):
            v = hg[p][j, pl.ds(_HID, 16)] + gb[p][j] + aev[p][pl.ds(j * 16, 16)]
            v = jnp.where(v >= 0.0, v, 0.2 * v)
            w = jnp.exp(v)
            mb[p][j, pl.ds(_HID, 16)] = w
            for hh in range(_H):
                lane = jnp.full((16,), hh, jnp.int32)
                wsplat = w.at[lane].get(mode="promise_in_bounds")
                mb[p][j, pl.ds(hh * 16, 16)] = (
                    hg[p][j, pl.ds(hh * 16, 16)] * wsplat)

    for m in range(_NMEGA):
        r = row_base + m * _MEGA
        pltpu.sync_copy(eidx.at[pl.ds(r, _MEGA)], idxs)
        pltpu.sync_copy(eidx.at[pl.ds(_IDXROWS // 2 + r, _MEGA)], idxd)
        _stage(m, 0, 0)

        def _pair(u, carry):
            jj = 2 * u
            _wait_gathers(0)

            @pl.when(u > 0)
            def _():
                _wait_scatter(1)

            _stage(m, jj + 1, 1)

            @pl.when(u > 0)
            def _():
                _wait_scatter(0)

            _compute(0)
            _scatter(jj, 0)
            _wait_gathers(1)

            @pl.when(u < _MEGA // 2 - 1)
            def _():
                _stage(m, jj + 2, 0)

            _compute(1)
            _scatter(jj + 1, 1)
            return carry

        lax.fori_loop(0, _MEGA // 2, _pair, 0)
        _wait_scatter(0)
        _wait_scatter(1)

    plsc.subcore_barrier()
    for k in range(_RPT // _RB):
        pltpu.sync_copy(acc.at[pl.ds(r0 + k * _RB, _RB)], zb)
        pltpu.sync_copy(zb, out.at[c, pl.ds(r0 + k * _RB, _RB)])


def _final_body(x_ref, acc_ref, bias_ref, g_ref, b_ref, rep_ref, o_ref):
    a0 = acc_ref[0]
    a1 = acc_ref[1]
    msg = a0[:, :_HID] + a1[:, :_HID]
    den = a0[:, _HID:_HID + _H] + a1[:, _HID:_HID + _H]
    den128 = jnp.dot(den, rep_ref[...], preferred_element_type=jnp.float32)
    y = x_ref[...] + msg / (den128 + 1e-16) + bias_ref[...]
    mu = jnp.mean(y, axis=-1, keepdims=True)
    d = y - mu
    var = jnp.mean(d * d, axis=-1, keepdims=True)
    o_ref[...] = d / jnp.sqrt(var + 1e-5) * g_ref[...] + b_ref[...]


def kernel(x, edge_index, edge_attr, W, att_src, att_dst, W_e, att_edge, bias,
           ln_g, ln_b):
    f32 = jnp.float32
    sel = jnp.kron(jnp.eye(_H, dtype=f32), jnp.ones((_C, 1), f32))
    sel = jnp.pad(sel, ((0, 0), (0, 16 - _H)))
    rep = jnp.kron(jnp.eye(_H, dtype=f32), jnp.ones((1, _C), f32))
    asf = att_src.reshape(1, _HID)
    adf = att_dst.reshape(1, _HID)
    aef = att_edge.reshape(1, _HID)

    smat = jnp.kron(jnp.ones((_H, 1), f32), jnp.eye(16, dtype=f32))
    tmat = jnp.kron(jnp.ones((1, _H), f32), jnp.eye(16, dtype=f32))
    mmat = jnp.kron(jnp.eye(_H, dtype=f32), jnp.ones((16, 16), f32))
    ea2 = edge_attr.reshape(_E // 8, 128)
    bn = 400
    brr = (_E // 8) // (_N // bn)
    hx, sd, ae2 = pl.pallas_call(
        _dense_body,
        grid=(_N // bn,),
        in_specs=[
            pl.BlockSpec((bn, _HID), lambda i: (i, 0)),
            pl.BlockSpec((brr, 128), lambda i: (i, 0)),
            pl.BlockSpec((_HID, _HID), lambda i: (0, 0)),
            pl.BlockSpec((1, _HID), lambda i: (0, 0)),
            pl.BlockSpec((1, _HID), lambda i: (0, 0)),
            pl.BlockSpec((1, _HID), lambda i: (0, 0)),
            pl.BlockSpec((_ED, _HID), lambda i: (0, 0)),
            pl.BlockSpec((_HID, 16), lambda i: (0, 0)),
            pl.BlockSpec((_HID, 16), lambda i: (0, 0)),
            pl.BlockSpec((16, _HID), lambda i: (0, 0)),
            pl.BlockSpec((_HID, _HID), lambda i: (0, 0)),
        ],
        out_specs=[
            pl.BlockSpec((bn, _ROW), lambda i: (i, 0)),
            pl.BlockSpec((bn, 16), lambda i: (i, 0)),
            pl.BlockSpec((brr, 128), lambda i: (i, 0)),
        ],
        out_shape=[
            jax.ShapeDtypeStruct((_N, _ROW), f32),
            jax.ShapeDtypeStruct((_N, 16), f32),
            jax.ShapeDtypeStruct((_E // 8, 128), f32),
        ],
    )(x, ea2, W, asf, adf, aef, W_e, sel, smat, tmat, mmat)
    ae = ae2.reshape(-1)

    acc = _edge_pass(edge_index.reshape(_IDXROWS, _K), sd, ae, hx)

    bf = 1000
    y = pl.pallas_call(
        _final_body,
        grid=(_N // bf,),
        in_specs=[
            pl.BlockSpec((bf, _HID), lambda i: (i, 0)),
            pl.BlockSpec((2, bf, _ROW), lambda i: (0, i, 0)),
            pl.BlockSpec((1, _HID), lambda i: (0, 0)),
            pl.BlockSpec((1, _HID), lambda i: (0, 0)),
            pl.BlockSpec((1, _HID), lambda i: (0, 0)),
            pl.BlockSpec((_H, _HID), lambda i: (0, 0)),
        ],
        out_specs=pl.BlockSpec((bf, _HID), lambda i: (i, 0)),
        out_shape=jax.ShapeDtypeStruct((_N, _HID), f32),
    )(x, acc, bias.reshape(1, _HID), ln_g.reshape(1, _HID),
      ln_b.reshape(1, _HID), rep)
    return y


# speedup vs baseline: 1.9640x; 1.0094x over previous
import functools

import jax
import jax.numpy as jnp
import numpy as np
from jax import lax
from jax.experimental import pallas as pl
from jax.experimental.pallas import tpu as pltpu
from jax.experimental.pallas import tpu_sc as plsc

_N = 10000
_E = 320000
_HID = 128
_H = 8
_C = 16
_ED = 16
_ROW = 144
_NTILES = 32
_EPW = _E // _NTILES
_K = 40
_NCHUNK = _EPW // _K
_NP = 10240
_RPT = _NP // 16
_RB = 32


def _dense_body(x_ref, ea_ref, w_ref, asf_ref, adf_ref, aef_ref, we_ref,
                sel_ref, smat_ref, tmat_ref, mmat_ref, hx_ref, sd_ref, ae_ref):
    x = x_ref[...]
    h = jnp.dot(x, w_ref[...], preferred_element_type=jnp.float32)
    sel = sel_ref[...]
    hx_ref[:, : _HID] = h
    hx_ref[:, _HID:] = jnp.dot(h * asf_ref[...], sel,
                               preferred_element_type=jnp.float32)
    sd_ref[...] = jnp.dot(h * adf_ref[...], sel,
                          preferred_element_type=jnp.float32)
    amat = jnp.dot(we_ref[...] * aef_ref[...], sel_ref[...],
                   preferred_element_type=jnp.float32)
    bmat = jnp.dot(jnp.dot(smat_ref[...], amat, preferred_element_type=jnp.float32),
                   tmat_ref[...], preferred_element_type=jnp.float32) * mmat_ref[...]
    ae_ref[...] = jnp.dot(ea_ref[...], bmat, preferred_element_type=jnp.float32)


_mesh = plsc.VectorSubcoreMesh(core_axis_name="c", subcore_axis_name="s")


_MEGA = 50
_NMEGA = _EPW // (_MEGA * _K)
_IDXROWS = 2 * _E // _K


@functools.partial(
    pl.kernel,
    out_type=jax.ShapeDtypeStruct((2, _NP, _ROW), jnp.float32),
    mesh=_mesh,
    compiler_params=pltpu.CompilerParams(use_tc_tiling_on_sc=False),
    scratch_types=[
        pltpu.VMEM((_MEGA, _K), jnp.int32),
        pltpu.VMEM((_MEGA, _K), jnp.int32),
        pltpu.VMEM((_K, 16), jnp.float32),
        pltpu.VMEM((_K, 16), jnp.float32),
        pltpu.VMEM((_K * 16,), jnp.float32),
        pltpu.VMEM((_K * 16,), jnp.float32),
        pltpu.VMEM((_K, _ROW), jnp.float32),
        pltpu.VMEM((_K, _ROW), jnp.float32),
        pltpu.VMEM((_K, _ROW), jnp.float32),
        pltpu.VMEM((_K, _ROW), jnp.float32),
        pltpu.VMEM((_RB, _ROW), jnp.float32),
        pltpu.VMEM_SHARED((_NP, _ROW), jnp.float32),
        pltpu.SemaphoreType.DMA,
        pltpu.SemaphoreType.DMA,
        pltpu.SemaphoreType.DMA,
        pltpu.SemaphoreType.DMA,
    ],
)
def _edge_pass(eidx, sd, ae, hx, out,
               idxs, idxd, gb0, gb1, aev0, aev1,
               hg0, hg1, mb0, mb1, zb, acc, semg0, semg1, sems0, sems1):
    c = lax.axis_index("c")
    s = lax.axis_index("s")
    gwid = c * 16 + s
    gb = (gb0, gb1)
    aev = (aev0, aev1)
    hg = (hg0, hg1)
    mb = (mb0, mb1)
    semg = (semg0, semg1)
    sems = (sems0, sems1)
    zeros16 = jnp.zeros((16,), jnp.float32)

    def _zrow(r, carry):
        for cc in range(_ROW // 16):
            zb[r, pl.ds(cc * 16, 16)] = zeros16
        return carry

    lax.fori_loop(0, _RB, _zrow, 0)
    r0 = s * _RPT
    for k in range(_RPT // _RB):
        pltpu.sync_copy(zb, acc.at[pl.ds(r0 + k * _RB, _RB)])
    plsc.subcore_barrier()

    row_base = gwid * (_EPW // _K)

    def _stage(m, jj, p):
        base = (row_base + m * _MEGA + jj) * _K
        pltpu.async_copy(sd.at[idxd.at[jj]], gb[p], semg[p])
        pltpu.async_copy(hx.at[idxs.at[jj]], hg[p], semg[p])
        pltpu.async_copy(ae.at[pl.ds(base * 16, _K * 16)], aev[p], semg[p])

    def _wait_gathers(p):
        pltpu.make_async_copy(sd.at[idxd.at[0]], gb[p], semg[p]).wait()
        pltpu.make_async_copy(hx.at[idxs.at[0]], hg[p], semg[p]).wait()
        pltpu.make_async_copy(ae.at[pl.ds(0, _K * 16)], aev[p], semg[p]).wait()

    def _scatter(jj, p):
        pltpu.async_copy(mb[p], acc.at[idxd.at[jj]], sems[p], add=True)

    def _wait_scatter(p):
        pltpu.make_async_copy(mb[p], acc.at[idxd.at[0]], sems[p]).wait()

    def _compute(p):
        @plsc.parallel_loop(0, _K, 1, unroll=4)
        def _edge(j):
            v = hg[p][j, pl.ds(_HID, 16)] + gb[p][j] + aev[p][pl.ds(j * 16, 16)]
            v = jnp.where(v >= 0.0, v, 0.2 * v)
            w = jnp.exp(v)
            mb[p][j, pl.ds(_HID, 16)] = w
            for hh in range(_H):
                lane = jnp.full((16,), hh, jnp.int32)
                wsplat = w.at[lane].get(mode="promise_in_bounds")
                mb[p][j, pl.ds(hh * 16, 16)] = (
                    hg[p][j, pl.ds(hh * 16, 16)] * wsplat)

    for m in range(_NMEGA):
        r = row_base + m * _MEGA
        pltpu.sync_copy(eidx.at[pl.ds(r, _MEGA)], idxs)
        pltpu.sync_copy(eidx.at[pl.ds(_IDXROWS // 2 + r, _MEGA)], idxd)
        _stage(m, 0, 0)

        def _pair(u, carry):
            jj = 2 * u
            _wait_gathers(0)

            @pl.when(u > 0)
            def _():
                _wait_scatter(1)

            _stage(m, jj + 1, 1)

            @pl.when(u > 0)
            def _():
                _wait_scatter(0)

            _compute(0)
            _scatter(jj, 0)
            _wait_gathers(1)

            @pl.when(u < _MEGA // 2 - 1)
            def _():
                _stage(m, jj + 2, 0)

            _compute(1)
            _scatter(jj + 1, 1)
            return carry

        lax.fori_loop(0, _MEGA // 2, _pair, 0)
        _wait_scatter(0)
        _wait_scatter(1)

    plsc.subcore_barrier()
    for k in range(_RPT // _RB):
        pltpu.sync_copy(acc.at[pl.ds(r0 + k * _RB, _RB)], zb)
        pltpu.sync_copy(zb, out.at[c, pl.ds(r0 + k * _RB, _RB)])


def _final_body(x_ref, acc_ref, bias_ref, g_ref, b_ref, rep_ref, o_ref):
    a0 = acc_ref[0]
    a1 = acc_ref[1]
    msg = a0[:, :_HID] + a1[:, :_HID]
    den = a0[:, _HID:_HID + _H] + a1[:, _HID:_HID + _H]
    den128 = jnp.dot(den, rep_ref[...], preferred_element_type=jnp.float32)
    y = x_ref[...] + msg / (den128 + 1e-16) + bias_ref[...]
    mu = jnp.mean(y, axis=-1, keepdims=True)
    d = y - mu
    var = jnp.mean(d * d, axis=-1, keepdims=True)
    o_ref[...] = d / jnp.sqrt(var + 1e-5) * g_ref[...] + b_ref[...]


def kernel(x, edge_index, edge_attr, W, att_src, att_dst, W_e, att_edge, bias,
           ln_g, ln_b):
    f32 = jnp.float32
    sel = jnp.kron(jnp.eye(_H, dtype=f32), jnp.ones((_C, 1), f32))
    sel = jnp.pad(sel, ((0, 0), (0, 16 - _H)))
    rep = jnp.kron(jnp.eye(_H, dtype=f32), jnp.ones((1, _C), f32))
    asf = att_src.reshape(1, _HID)
    adf = att_dst.reshape(1, _HID)
    aef = att_edge.reshape(1, _HID)

    smat = jnp.kron(jnp.ones((_H, 1), f32), jnp.eye(16, dtype=f32))
    tmat = jnp.kron(jnp.ones((1, _H), f32), jnp.eye(16, dtype=f32))
    mmat = jnp.kron(jnp.eye(_H, dtype=f32), jnp.ones((16, 16), f32))
    ea2 = edge_attr.reshape(_E // 8, 128)
    bn = 400
    brr = (_E // 8) // (_N // bn)
    hx, sd, ae2 = pl.pallas_call(
        _dense_body,
        grid=(_N // bn,),
        in_specs=[
            pl.BlockSpec((bn, _HID), lambda i: (i, 0)),
            pl.BlockSpec((brr, 128), lambda i: (i, 0)),
            pl.BlockSpec((_HID, _HID), lambda i: (0, 0)),
            pl.BlockSpec((1, _HID), lambda i: (0, 0)),
            pl.BlockSpec((1, _HID), lambda i: (0, 0)),
            pl.BlockSpec((1, _HID), lambda i: (0, 0)),
            pl.BlockSpec((_ED, _HID), lambda i: (0, 0)),
            pl.BlockSpec((_HID, 16), lambda i: (0, 0)),
            pl.BlockSpec((_HID, 16), lambda i: (0, 0)),
            pl.BlockSpec((16, _HID), lambda i: (0, 0)),
            pl.BlockSpec((_HID, _HID), lambda i: (0, 0)),
        ],
        out_specs=[
            pl.BlockSpec((bn, _ROW), lambda i: (i, 0)),
            pl.BlockSpec((bn, 16), lambda i: (i, 0)),
            pl.BlockSpec((brr, 128), lambda i: (i, 0)),
        ],
        out_shape=[
            jax.ShapeDtypeStruct((_N, _ROW), f32),
            jax.ShapeDtypeStruct((_N, 16), f32),
            jax.ShapeDtypeStruct((_E // 8, 128), f32),
        ],
    )(x, ea2, W, asf, adf, aef, W_e, sel, smat, tmat, mmat)
    ae = ae2.reshape(-1)

    acc = _edge_pass(edge_index.reshape(_IDXROWS, _K), sd, ae, hx)

    bf = 1000
    y = pl.pallas_call(
        _final_body,
        grid=(_N // bf,),
        in_specs=[
            pl.BlockSpec((bf, _HID), lambda i: (i, 0)),
            pl.BlockSpec((2, bf, _ROW), lambda i: (0, i, 0)),
            pl.BlockSpec((1, _HID), lambda i: (0, 0)),
            pl.BlockSpec((1, _HID), lambda i: (0, 0)),
            pl.BlockSpec((1, _HID), lambda i: (0, 0)),
            pl.BlockSpec((_H, _HID), lambda i: (0, 0)),
        ],
        out_specs=pl.BlockSpec((bf, _HID), lambda i: (i, 0)),
        out_shape=jax.ShapeDtypeStruct((_N, _HID), f32),
    )(x, acc, bias.reshape(1, _HID), ln_g.reshape(1, _HID),
      ln_b.reshape(1, _HID), rep)
    return y
